# baseline (speedup 1.0000x reference)
import functools

import jax
import jax.numpy as jnp
from jax import lax
from jax.experimental import pallas as pl
from jax.experimental.pallas import tpu as pltpu
from jax.experimental.pallas import tpu_sc as plsc

_FIELD0 = 1000000
_EMBED = 32
_BATCH = 16384
_NC = 2
_NS = 16
_NW = _NC * _NS
_BPW = _BATCH // _NW
_CHUNK = 128
_NCHUNK = _BPW // _CHUNK


def _gmf_body(u_hbm, it_hbm, table_hbm, wb_hbm, out_hbm,
              uidx_v, itidx_v, urows_v, itrows_v, out_v, wb_v, sem):
    wid = lax.axis_index("s") * _NC + lax.axis_index("c")
    row0 = wid * _NCHUNK

    pltpu.sync_copy(u_hbm.at[pl.ds(row0, _NCHUNK)], uidx_v)
    pltpu.sync_copy(it_hbm.at[pl.ds(row0, _NCHUNK)], itidx_v)
    pltpu.sync_copy(wb_hbm, wb_v)

    for j in range(_NCHUNK):
        for k in range(_CHUNK // 16):
            sl = pl.ds(k * 16, 16)
            itidx_v[j, sl] = itidx_v[j, sl] + _FIELD0

    copies = []
    for j in range(_NCHUNK):
        dst = urows_v.at[pl.ds(j * _CHUNK, _CHUNK)]
        copies.append(pltpu.async_copy(table_hbm.at[uidx_v.at[j]], dst, sem))
        dst = itrows_v.at[pl.ds(j * _CHUNK, _CHUNK)]
        copies.append(pltpu.async_copy(table_hbm.at[itidx_v.at[j]], dst, sem))
    for c in copies:
        c.wait()

    w0 = wb_v[pl.ds(0, 16)]
    w1 = wb_v[pl.ds(16, 16)]
    bvec = wb_v[pl.ds(32, 16)]
    lanes = lax.iota(jnp.int32, 16)

    def body(blk, _):
        i0 = blk * 16
        res = jnp.zeros((16,), jnp.float32)
        for r in range(16):
            i = i0 + r
            u0 = urows_v[i, pl.ds(0, 16)]
            u1 = urows_v[i, pl.ds(16, 16)]
            t0 = itrows_v[i, pl.ds(0, 16)]
            t1 = itrows_v[i, pl.ds(16, 16)]
            s = u0 * t0 * w0 + u1 * t1 * w1
            res = jnp.where(lanes == r, jnp.sum(s), res)
        out_v[pl.ds(i0, 16)] = jnp.maximum(res + bvec, 0.0)
        return 0

    lax.fori_loop(0, _BPW // 16, body, 0)

    pltpu.sync_copy(out_v, out_hbm.at[pl.ds(wid * _BPW, _BPW)])


@jax.jit
def kernel(x, table, W, b):
    u = x[:, 0].reshape(_NW * _NCHUNK, _CHUNK)
    it = x[:, 1].reshape(_NW * _NCHUNK, _CHUNK)
    wb = jnp.concatenate([W.reshape(-1), jnp.broadcast_to(b, (16,))])

    mesh = plsc.VectorSubcoreMesh(core_axis_name="c", subcore_axis_name="s")
    run = pl.kernel(
        _gmf_body,
        mesh=mesh,
        compiler_params=pltpu.CompilerParams(
            needs_layout_passes=False, use_tc_tiling_on_sc=False),
        out_type=jax.ShapeDtypeStruct((_BATCH,), jnp.float32),
        scratch_types=[
            pltpu.VMEM((_NCHUNK, _CHUNK), jnp.int32),
            pltpu.VMEM((_NCHUNK, _CHUNK), jnp.int32),
            pltpu.VMEM((_BPW, _EMBED), jnp.float32),
            pltpu.VMEM((_BPW, _EMBED), jnp.float32),
            pltpu.VMEM((_BPW,), jnp.float32),
            pltpu.VMEM((48,), jnp.float32),
            pltpu.SemaphoreType.DMA,
        ],
    )
    y = run(u, it, table, wb)
    return y.reshape(_BATCH, 1)
